# baseline (speedup 1.0000x reference)
import jax
import jax.numpy as jnp
from jax.experimental import pallas as pl


def _add_kernel(x_ref, emb_ref, o_ref):
    o_ref[...] = x_ref[...] + emb_ref[...]


def kernel(x, emb):
    B, S, D = x.shape
    S_BLK = 256
    assert S % S_BLK == 0
    emb_s = jax.lax.slice(emb, (0, 0), (S, D))
    return pl.pallas_call(
        _add_kernel,
        grid=(S // S_BLK,),
        in_specs=[
            pl.BlockSpec((B, S_BLK, D), lambda i: (0, i, 0)),
            pl.BlockSpec((S_BLK, D), lambda i: (i, 0)),
        ],
        out_specs=pl.BlockSpec((B, S_BLK, D), lambda i: (0, i, 0)),
        out_shape=jax.ShapeDtypeStruct((B, S, D), x.dtype),
    )(x, emb_s)


# speedup vs baseline: 1.0322x; 1.0322x over previous
import jax
import jax.numpy as jnp
from jax.experimental import pallas as pl


def _add_kernel(x_ref, emb_ref, o_ref):
    o_ref[...] = x_ref[...] + emb_ref[...]


def kernel(x, emb):
    B, S, D = x.shape
    S_BLK = 1024
    assert S % S_BLK == 0
    emb_s = jax.lax.slice(emb, (0, 0), (S, D))
    return pl.pallas_call(
        _add_kernel,
        grid=(S // S_BLK,),
        in_specs=[
            pl.BlockSpec((B, S_BLK, D), lambda i: (0, i, 0)),
            pl.BlockSpec((S_BLK, D), lambda i: (i, 0)),
        ],
        out_specs=pl.BlockSpec((B, S_BLK, D), lambda i: (0, i, 0)),
        out_shape=jax.ShapeDtypeStruct((B, S, D), x.dtype),
    )(x, emb_s)
